# baseline (speedup 1.0000x reference)
import jax
import jax.numpy as jnp
from jax import lax
from jax.experimental import pallas as pl
from jax.experimental.pallas import tpu as pltpu
from jax.experimental.pallas import tpu_sc as plsc

N = 50000
E = 800000
H = 4
C = 16
D = 64
G = 128
V = 256
OUT = 128

NC = 2
NS = 16
L = 16
NW = NC * NS

EP = 851968
K = 128
S1_CHUNKS = EP // NW // K
S2_CHUNKS = EP // NS // K

HALF = N // 2
HALFP = HALF + 88
NZP = N + 48
NP32 = 53248
GP = G + 8

D2 = D // 2
R = 1000
NB = N // R
NEG = -3.0e38


def _f32(v):
    return v.astype(jnp.float32)


def _head_sum_matrix():
    f = lax.broadcasted_iota(jnp.int32, (D, H), 0) // C
    hh = lax.broadcasted_iota(jnp.int32, (D, H), 1)
    return _f32(f == hh)


def _expand_matrix():
    hh = lax.broadcasted_iota(jnp.int32, (H, D), 0)
    f = lax.broadcasted_iota(jnp.int32, (H, D), 1) // C
    return _f32(f == hh)


def _col_place(n, m, at):
    i = lax.broadcasted_iota(jnp.int32, (n, m), 0)
    j = lax.broadcasted_iota(jnp.int32, (n, m), 1)
    return _f32(j == i + at)


def _xw_al_ar(h, w_ref, asf_ref, adf_ref):
    xw = jnp.dot(h, w_ref[...], preferred_element_type=jnp.float32)
    m = _head_sum_matrix()
    al = jnp.dot(xw * asf_ref[...], m, preferred_element_type=jnp.float32)
    ar = jnp.dot(xw * adf_ref[...], m, preferred_element_type=jnp.float32)
    alr = (jnp.dot(jnp.exp(al), _col_place(H, 16, 0),
                   preferred_element_type=jnp.float32)
           + jnp.dot(jnp.exp(0.2 * al), _col_place(H, 16, 4),
                     preferred_element_type=jnp.float32)
           + jnp.dot(jnp.exp(ar), _col_place(H, 16, 8),
                     preferred_element_type=jnp.float32)
           + jnp.dot(jnp.exp(0.2 * ar), _col_place(H, 16, 12),
                     preferred_element_type=jnp.float32))
    return xw, alr


def _k1_body(x_ref, emb_ref, w_ref, asf_ref, adf_ref, h_ref, xw_ref, alr_ref):
    ids = x_ref[0, 0, :]
    oh = _f32(ids[:, None] == lax.broadcasted_iota(jnp.int32, (R, V), 1))
    h = jnp.dot(oh, emb_ref[...], preferred_element_type=jnp.float32)
    xw, alr = _xw_al_ar(h, w_ref, asf_ref, adf_ref)
    h_ref[...] = h
    xw_ref[0] = xw[:, 0:D2]
    xw_ref[1] = xw[:, D2:D]
    alr_ref[...] = alr


def _k1(x3d, emb, w0, asf, adf):
    return pl.pallas_call(
        _k1_body,
        grid=(NB,),
        in_specs=[
            pl.BlockSpec((1, 1, R), lambda b: (b, 0, 0)),
            pl.BlockSpec((V, D), lambda b: (0, 0)),
            pl.BlockSpec((D, D), lambda b: (0, 0)),
            pl.BlockSpec((1, D), lambda b: (0, 0)),
            pl.BlockSpec((1, D), lambda b: (0, 0)),
        ],
        out_specs=[
            pl.BlockSpec((R, D), lambda b: (b, 0)),
            pl.BlockSpec((2, R, D2), lambda b: (0, b, 0)),
            pl.BlockSpec((R, 16), lambda b: (b, 0)),
        ],
        out_shape=[
            jax.ShapeDtypeStruct((N, D), jnp.float32),
            jax.ShapeDtypeStruct((2, N, D2), jnp.float32),
            jax.ShapeDtypeStruct((N, 16), jnp.float32),
        ],
    )(x3d, emb, w0, asf, adf)


def _vgather(v, idx16):
    return lax.gather(
        v, idx16[:, None],
        lax.GatherDimensionNumbers(offset_dims=(), collapsed_slice_dims=(0,),
                                   start_index_map=(0,)),
        (1,), mode=lax.GatherScatterMode.PROMISE_IN_BOUNDS)


S1_PAIRS = S1_CHUNKS // 2
S2_PAIRS = S2_CHUNKS // 2


def _s1_body(src_hbm, dst_hbm, alr_hbm, zz_hbm, p_hbm, zout_hbm,
             zsp,
             src_a, dst_a, dstc_a, ga_a, gb_a, p_a,
             src_b, dst_b, dstc_b, ga_b, gb_b, p_b,
             lsem_a, lsem_b, gsem_a, gsem_b):
    c = lax.axis_index("c")
    s = lax.axis_index("s")
    i16 = lax.iota(jnp.int32, L)
    shf8 = jnp.bitwise_and(i16 + 8, 15)
    shf4 = jnp.bitwise_and(i16 + 4, 15)
    zrows = NZP // NS
    pltpu.sync_copy(zz_hbm.at[pl.ds(s * zrows, zrows)],
                    zsp.at[pl.ds(s * zrows, zrows)])
    plsc.subcore_barrier()
    base_w = c * (EP // NC) + s * (EP // NW)

    A = (src_a, dst_a, dstc_a, ga_a, gb_a, p_a, lsem_a, gsem_a)
    B = (src_b, dst_b, dstc_b, ga_b, gb_b, p_b, lsem_b, gsem_b)

    def lin_issue(g, bufs):
        sv, dv = bufs[0], bufs[1]
        base = base_w + g * K
        pltpu.async_copy(src_hbm.at[pl.ds(base, K)], sv, bufs[6])
        pltpu.async_copy(dst_hbm.at[pl.ds(base, K)], dv, bufs[6])

    def lin_wait(g, bufs):
        sv, dv = bufs[0], bufs[1]
        base = base_w + g * K
        pltpu.make_async_copy(src_hbm.at[pl.ds(base, K)], sv, bufs[6]).wait()
        pltpu.make_async_copy(dst_hbm.at[pl.ds(base, K)], dv, bufs[6]).wait()

    def gat_issue(bufs):
        sv, dv, dcv, ga, gb = bufs[0], bufs[1], bufs[2], bufs[3], bufs[4]
        for i in range(K // L):
            dcv[pl.ds(i * L, L)] = jnp.minimum(dv[pl.ds(i * L, L)], N - 1)
        pltpu.async_copy(alr_hbm.at[sv], ga, bufs[7])
        pltpu.async_copy(alr_hbm.at[dcv], gb, bufs[7])

    def gat_wait(bufs):
        sv, dcv, ga, gb = bufs[0], bufs[2], bufs[3], bufs[4]
        pltpu.make_async_copy(alr_hbm.at[sv], ga, bufs[7]).wait()
        pltpu.make_async_copy(alr_hbm.at[dcv], gb, bufs[7]).wait()

    def compute(g, bufs):
        dv, ga, gb, pv = bufs[1], bufs[3], bufs[4], bufs[5]
        base = base_w + g * K

        def edge(e, cc):
            prod = ga[e, :] * _vgather(gb[e, :], shf8)
            pv[e, :] = jnp.maximum(prod, _vgather(prod, shf4))
            return cc

        lax.fori_loop(0, K, edge, 0, unroll=4)
        pltpu.sync_copy(pv, p_hbm.at[pl.ds(base, K)])
        pltpu.sync_copy(pv, zsp.at[dv], add=True)

    lin_issue(0, A)
    lin_wait(0, A)
    gat_issue(A)
    lin_issue(1, B)

    def pair(i, carry):
        g0 = 2 * i
        lin_wait(g0 + 1, B)
        gat_issue(B)
        gat_wait(A)
        compute(g0, A)
        lin_issue(g0 + 2, A)
        gat_wait(B)
        compute(g0 + 1, B)
        lin_wait(g0 + 2, A)
        gat_issue(A)
        lin_issue(g0 + 3, B)
        return carry

    lax.fori_loop(0, S1_PAIRS - 1, pair, 0)
    g0 = S1_CHUNKS - 2
    lin_wait(g0 + 1, B)
    gat_issue(B)
    gat_wait(A)
    compute(g0, A)
    gat_wait(B)
    compute(g0 + 1, B)

    plsc.subcore_barrier()
    pltpu.sync_copy(zsp.at[pl.ds(s * zrows, zrows)],
                    zout_hbm.at[c, pl.ds(s * zrows, zrows)])


def _s1(srcp, dstp, alr, zz16):
    f = pl.kernel(
        _s1_body,
        out_type=[
            jax.ShapeDtypeStruct((EP, 16), jnp.float32),
            jax.ShapeDtypeStruct((NC, NZP, 16), jnp.float32),
        ],
        mesh=plsc.VectorSubcoreMesh(core_axis_name="c", subcore_axis_name="s"),
        compiler_params=pltpu.CompilerParams(use_tc_tiling_on_sc=False),
        scratch_types=(
            [pltpu.VMEM_SHARED((NZP, 16), jnp.float32)]
            + 2 * [pltpu.VMEM((K,), jnp.int32),
                   pltpu.VMEM((K,), jnp.int32),
                   pltpu.VMEM((K,), jnp.int32),
                   pltpu.VMEM((K, 16), jnp.float32),
                   pltpu.VMEM((K, 16), jnp.float32),
                   pltpu.VMEM((K, 16), jnp.float32)]
            + 4 * [pltpu.SemaphoreType.DMA]
        ),
    )
    return f(srcp, dstp, alr, zz16)


def _s2_body(src_hbm, dst_hbm, p_hbm, xw_hbm, zu_hbm, uout_hbm,
             usp,
             src_a, dst_a, idxg_a, p_a, xw_a,
             src_b, dst_b, idxg_b, p_b, xw_b,
             lsem_a, lsem_b, gsem_a, gsem_b):
    c = lax.axis_index("c")
    s = lax.axis_index("s")
    urows = NZP // NS
    pltpu.sync_copy(zu_hbm.at[pl.ds(s * urows, urows)],
                    usp.at[pl.ds(s * urows, urows)])
    plsc.subcore_barrier()
    base_w = s * (EP // NS)
    rowoff = c * N

    A = (src_a, dst_a, idxg_a, p_a, xw_a, lsem_a, gsem_a)
    B = (src_b, dst_b, idxg_b, p_b, xw_b, lsem_b, gsem_b)

    def lin_issue(g, bufs):
        sv, dv, pv = bufs[0], bufs[1], bufs[3]
        base = base_w + g * K
        pltpu.async_copy(src_hbm.at[pl.ds(base, K)], sv, bufs[5])
        pltpu.async_copy(dst_hbm.at[pl.ds(base, K)], dv, bufs[5])
        pltpu.async_copy(p_hbm.at[pl.ds(base, K)], pv, bufs[5])

    def lin_wait(g, bufs):
        sv, dv, pv = bufs[0], bufs[1], bufs[3]
        base = base_w + g * K
        pltpu.make_async_copy(src_hbm.at[pl.ds(base, K)], sv, bufs[5]).wait()
        pltpu.make_async_copy(dst_hbm.at[pl.ds(base, K)], dv, bufs[5]).wait()
        pltpu.make_async_copy(p_hbm.at[pl.ds(base, K)], pv, bufs[5]).wait()

    def gat_issue(bufs):
        sv, gv = bufs[0], bufs[2]
        for i in range(K // L):
            gv[pl.ds(i * L, L)] = sv[pl.ds(i * L, L)] + rowoff
        pltpu.async_copy(xw_hbm.at[gv], bufs[4], bufs[6])

    def gat_wait(bufs):
        pltpu.make_async_copy(xw_hbm.at[bufs[2]], bufs[4], bufs[6]).wait()

    ph0 = jnp.full((L,), 2 * c, jnp.int32)
    ph1 = ph0 + 1

    def compute(bufs):
        dv, pv, xv = bufs[1], bufs[3], bufs[4]

        def edge(e, cc):
            prow = pv[e, :]
            pk0 = _vgather(prow, ph0)
            pk1 = _vgather(prow, ph1)
            xv[e, pl.ds(0, L)] = xv[e, pl.ds(0, L)] * pk0
            xv[e, pl.ds(L, L)] = xv[e, pl.ds(L, L)] * pk1
            return cc

        lax.fori_loop(0, K, edge, 0, unroll=4)
        pltpu.sync_copy(xv, usp.at[dv], add=True)

    lin_issue(0, A)
    lin_wait(0, A)
    gat_issue(A)
    lin_issue(1, B)

    def pair(i, carry):
        g0 = 2 * i
        lin_wait(g0 + 1, B)
        gat_issue(B)
        gat_wait(A)
        compute(A)
        lin_issue(g0 + 2, A)
        gat_wait(B)
        compute(B)
        lin_wait(g0 + 2, A)
        gat_issue(A)
        lin_issue(g0 + 3, B)
        return carry

    lax.fori_loop(0, S2_PAIRS - 1, pair, 0)
    lin_wait(S2_CHUNKS - 1, B)
    gat_issue(B)
    gat_wait(A)
    compute(A)
    gat_wait(B)
    compute(B)

    plsc.subcore_barrier()
    pltpu.sync_copy(usp.at[pl.ds(s * urows, urows)],
                    uout_hbm.at[c, pl.ds(s * urows, urows)])


def _s2(srcp, dstp, p, xw2, zu):
    f = pl.kernel(
        _s2_body,
        out_type=[
            jax.ShapeDtypeStruct((NC, NZP, D2), jnp.float32),
        ],
        mesh=plsc.VectorSubcoreMesh(core_axis_name="c", subcore_axis_name="s"),
        compiler_params=pltpu.CompilerParams(use_tc_tiling_on_sc=False),
        scratch_types=(
            [pltpu.VMEM_SHARED((NZP, D2), jnp.float32)]
            + 2 * [pltpu.VMEM((K,), jnp.int32),
                   pltpu.VMEM((K,), jnp.int32),
                   pltpu.VMEM((K,), jnp.int32),
                   pltpu.VMEM((K, 16), jnp.float32),
                   pltpu.VMEM((K, D2), jnp.float32)]
            + 4 * [pltpu.SemaphoreType.DMA]
        ),
    )
    return f(srcp, dstp, p, xw2, zu)


def _stats_body(u0_ref, u1_ref, z_ref, res_ref, bias_ref, batch_ref,
                t_ref, stats_ref, acc):
    b = pl.program_id(0)
    zs = z_ref[0, :, 0:H] + z_ref[1, :, 0:H]
    rz = 1.0 / (zs + 1e-16)
    rzx = jnp.dot(rz, _expand_matrix(), preferred_element_type=jnp.float32)
    t0 = u0_ref[0] * rzx[:, 0:D2] + bias_ref[..., 0:D2] + res_ref[:, 0:D2]
    t1 = u1_ref[0] * rzx[:, D2:D] + bias_ref[..., D2:D] + res_ref[:, D2:D]
    t_ref[:, 0:D2] = t0
    t_ref[:, D2:D] = t1
    t = jnp.concatenate([t0, t1], axis=1)

    ids = batch_ref[0, 0, :]
    oh = _f32(ids[:, None] == lax.broadcasted_iota(jnp.int32, (R, G), 1))
    ones = jnp.ones((R, 1), jnp.float32)
    a = jnp.sum(t, axis=1, keepdims=True)
    q = jnp.sum(t * t, axis=1, keepdims=True)
    cols = (jnp.dot(ones, _col_place(1, 8, 0), preferred_element_type=jnp.float32)
            + jnp.dot(a, _col_place(1, 8, 1), preferred_element_type=jnp.float32)
            + jnp.dot(q, _col_place(1, 8, 2), preferred_element_type=jnp.float32))
    blk = lax.dot_general(oh, cols, (((0,), (0,)), ((), ())),
                          preferred_element_type=jnp.float32)

    @pl.when(b == 0)
    def _():
        acc[...] = jnp.zeros_like(acc)

    acc[...] += blk

    @pl.when(b == NB - 1)
    def _():
        stats_ref[...] = acc[...]


def _k_stats(uout, zout, res, bias1x, batch3d):
    return pl.pallas_call(
        _stats_body,
        grid=(NB,),
        in_specs=[
            pl.BlockSpec((1, R, D2), lambda b: (0, b, 0)),
            pl.BlockSpec((1, R, D2), lambda b: (1, b, 0)),
            pl.BlockSpec((NC, R, 16), lambda b: (0, b, 0)),
            pl.BlockSpec((R, D), lambda b: (b, 0)),
            pl.BlockSpec((1, D), lambda b: (0, 0)),
            pl.BlockSpec((1, 1, R), lambda b: (b, 0, 0)),
        ],
        out_specs=[
            pl.BlockSpec((R, D), lambda b: (b, 0)),
            pl.BlockSpec((G, 8), lambda b: (0, 0)),
        ],
        out_shape=[
            jax.ShapeDtypeStruct((N, D), jnp.float32),
            jax.ShapeDtypeStruct((G, 8), jnp.float32),
        ],
        scratch_shapes=[pltpu.VMEM((G, 8), jnp.float32)],
    )(uout, uout, zout, res, bias1x, batch3d)


def _ln_rows(t, ids, stats_ref, lnw_ref, lnb_ref):
    st = stats_ref[...]
    cnt = st[:, 0:1]
    norm = jnp.maximum(cnt, 1.0) * D
    mean = st[:, 1:2] / norm
    var = st[:, 2:3] / norm - mean * mean
    inv = 1.0 / jnp.sqrt(var + 1e-5)
    gv = (jnp.dot(mean, _col_place(1, 8, 0), preferred_element_type=jnp.float32)
          + jnp.dot(inv, _col_place(1, 8, 1), preferred_element_type=jnp.float32))
    oh = _f32(ids[:, None] == lax.broadcasted_iota(jnp.int32, (R, G), 1))
    mv = jnp.dot(oh, gv, preferred_element_type=jnp.float32)
    return ((t - mv[:, 0:1]) * mv[:, 1:2]) * lnw_ref[...] + lnb_ref[...], oh


def _ln_map_body(t_ref, batch_ref, stats_ref, lnw_ref, lnb_ref,
                 w_ref, asf_ref, adf_ref, h_ref, xw_ref, alr_ref):
    hn, _ = _ln_rows(t_ref[...], batch_ref[0, 0, :], stats_ref, lnw_ref, lnb_ref)
    xw, alr = _xw_al_ar(hn, w_ref, asf_ref, adf_ref)
    h_ref[...] = hn
    xw_ref[0] = xw[:, 0:D2]
    xw_ref[1] = xw[:, D2:D]
    alr_ref[...] = alr


def _k_ln_map(t, batch3d, stats, lnw1x, lnb1x, w1, asf, adf):
    return pl.pallas_call(
        _ln_map_body,
        grid=(NB,),
        in_specs=[
            pl.BlockSpec((R, D), lambda b: (b, 0)),
            pl.BlockSpec((1, 1, R), lambda b: (b, 0, 0)),
            pl.BlockSpec((G, 8), lambda b: (0, 0)),
            pl.BlockSpec((1, D), lambda b: (0, 0)),
            pl.BlockSpec((1, D), lambda b: (0, 0)),
            pl.BlockSpec((D, D), lambda b: (0, 0)),
            pl.BlockSpec((1, D), lambda b: (0, 0)),
            pl.BlockSpec((1, D), lambda b: (0, 0)),
        ],
        out_specs=[
            pl.BlockSpec((R, D), lambda b: (b, 0)),
            pl.BlockSpec((2, R, D2), lambda b: (0, b, 0)),
            pl.BlockSpec((R, 16), lambda b: (b, 0)),
        ],
        out_shape=[
            jax.ShapeDtypeStruct((N, D), jnp.float32),
            jax.ShapeDtypeStruct((2, N, D2), jnp.float32),
            jax.ShapeDtypeStruct((N, 16), jnp.float32),
        ],
    )(t, batch3d, stats, lnw1x, lnb1x, w1, asf, adf)


def _ln_pool_body(t_ref, batch_ref, stats_ref, lnw_ref, lnb_ref,
                  h_ref, spool_ref, acc):
    b = pl.program_id(0)
    hn, oh = _ln_rows(t_ref[...], batch_ref[0, 0, :], stats_ref,
                      lnw_ref, lnb_ref)
    h_ref[...] = hn
    blk = lax.dot_general(oh, hn, (((0,), (0,)), ((), ())),
                          preferred_element_type=jnp.float32)

    @pl.when(b == 0)
    def _():
        acc[...] = jnp.zeros_like(acc)

    acc[...] += blk

    @pl.when(b == NB - 1)
    def _():
        spool_ref[...] = acc[...]


def _k_ln_pool(t, batch3d, stats, lnw1x, lnb1x):
    return pl.pallas_call(
        _ln_pool_body,
        grid=(NB,),
        in_specs=[
            pl.BlockSpec((R, D), lambda b: (b, 0)),
            pl.BlockSpec((1, 1, R), lambda b: (b, 0, 0)),
            pl.BlockSpec((G, 8), lambda b: (0, 0)),
            pl.BlockSpec((1, D), lambda b: (0, 0)),
            pl.BlockSpec((1, D), lambda b: (0, 0)),
        ],
        out_specs=[
            pl.BlockSpec((R, D), lambda b: (b, 0)),
            pl.BlockSpec((G, D), lambda b: (0, 0)),
        ],
        out_shape=[
            jax.ShapeDtypeStruct((NP32, D), jnp.float32),
            jax.ShapeDtypeStruct((G, D), jnp.float32),
        ],
        scratch_shapes=[pltpu.VMEM((G, D), jnp.float32)],
    )(t, batch3d, stats, lnw1x, lnb1x)


def _phase0(u0_ref, u1_ref, z_ref, res_ref, bias_ref, batch_ref, t_ref, acc):
    b = pl.program_id(0)
    zs = z_ref[0, :, 0:H] + z_ref[1, :, 0:H]
    rz = 1.0 / (zs + 1e-16)
    rzx = jnp.dot(rz, _expand_matrix(), preferred_element_type=jnp.float32)
    t0 = u0_ref[0] * rzx[:, 0:D2] + bias_ref[..., 0:D2] + res_ref[:, 0:D2]
    t1 = u1_ref[0] * rzx[:, D2:D] + bias_ref[..., D2:D] + res_ref[:, D2:D]
    t_ref[:, 0:D2] = t0
    t_ref[:, D2:D] = t1
    t = jnp.concatenate([t0, t1], axis=1)
    ids = batch_ref[0, 0, :]
    oh = _f32(ids[:, None] == lax.broadcasted_iota(jnp.int32, (R, G), 1))
    ones = jnp.ones((R, 1), jnp.float32)
    a = jnp.sum(t, axis=1, keepdims=True)
    q = jnp.sum(t * t, axis=1, keepdims=True)
    cols = (jnp.dot(ones, _col_place(1, 8, 0), preferred_element_type=jnp.float32)
            + jnp.dot(a, _col_place(1, 8, 1), preferred_element_type=jnp.float32)
            + jnp.dot(q, _col_place(1, 8, 2), preferred_element_type=jnp.float32))
    blk = lax.dot_general(oh, cols, (((0,), (0,)), ((), ())),
                          preferred_element_type=jnp.float32)

    @pl.when(b == 0)
    def _():
        acc[...] = jnp.zeros_like(acc)

    acc[...] += blk


def _fused_map_body(u0_ref, u1_ref, z_ref, res_ref, bias_ref, batch_ref,
                    lnw_ref, lnb_ref, w_ref, asf_ref, adf_ref,
                    t_ref, h_ref, xw_ref, alr_ref, acc):
    b = pl.program_id(0)

    @pl.when(b < NB)
    def _():
        _phase0(u0_ref, u1_ref, z_ref, res_ref, bias_ref, batch_ref, t_ref, acc)

    @pl.when(b >= NB)
    def _():
        hn, _ = _ln_rows(t_ref[...], batch_ref[0, 0, :], acc, lnw_ref, lnb_ref)
        xw, alr = _xw_al_ar(hn, w_ref, asf_ref, adf_ref)
        h_ref[...] = hn
        xw_ref[0] = xw[:, 0:D2]
        xw_ref[1] = xw[:, D2:D]
        alr_ref[...] = alr


def _k_fused_map(uout, zout, res, bias1x, batch3d, lnw1x, lnb1x, w1, asf, adf):
    m = NB
    return pl.pallas_call(
        _fused_map_body,
        grid=(2 * NB,),
        in_specs=[
            pl.BlockSpec((1, R, D2), lambda b: (0, b % m, 0)),
            pl.BlockSpec((1, R, D2), lambda b: (1, b % m, 0)),
            pl.BlockSpec((NC, R, 16), lambda b: (0, b % m, 0)),
            pl.BlockSpec((R, D), lambda b: (b % m, 0)),
            pl.BlockSpec((1, D), lambda b: (0, 0)),
            pl.BlockSpec((1, 1, R), lambda b: (b % m, 0, 0)),
            pl.BlockSpec((1, D), lambda b: (0, 0)),
            pl.BlockSpec((1, D), lambda b: (0, 0)),
            pl.BlockSpec((D, D), lambda b: (0, 0)),
            pl.BlockSpec((1, D), lambda b: (0, 0)),
            pl.BlockSpec((1, D), lambda b: (0, 0)),
        ],
        out_specs=[
            pl.BlockSpec((R, D), lambda b: (b % m, 0)),
            pl.BlockSpec((R, D), lambda b: (b % m, 0)),
            pl.BlockSpec((2, R, D2), lambda b: (0, b % m, 0)),
            pl.BlockSpec((R, 16), lambda b: (b % m, 0)),
        ],
        out_shape=[
            jax.ShapeDtypeStruct((N, D), jnp.float32),
            jax.ShapeDtypeStruct((N, D), jnp.float32),
            jax.ShapeDtypeStruct((2, N, D2), jnp.float32),
            jax.ShapeDtypeStruct((N, 16), jnp.float32),
        ],
        scratch_shapes=[pltpu.VMEM((G, 8), jnp.float32)],
    )(uout, uout, zout, res, bias1x, batch3d, lnw1x, lnb1x, w1, asf, adf)


def _fused_pool_body(u0_ref, u1_ref, z_ref, res_ref, bias_ref, batch_ref,
                     lnw_ref, lnb_ref,
                     t_ref, h_ref, spool_ref, stats_ref, acc, acc2):
    b = pl.program_id(0)

    @pl.when(b < NB)
    def _():
        _phase0(u0_ref, u1_ref, z_ref, res_ref, bias_ref, batch_ref, t_ref, acc)

    @pl.when(b >= NB)
    def _():
        hn, oh = _ln_rows(t_ref[...], batch_ref[0, 0, :], acc, lnw_ref, lnb_ref)
        h_ref[...] = hn
        blk = lax.dot_general(oh, hn, (((0,), (0,)), ((), ())),
                              preferred_element_type=jnp.float32)

        @pl.when(b == NB)
        def _():
            acc2[...] = jnp.zeros_like(acc2)

        acc2[...] += blk

        @pl.when(b == 2 * NB - 1)
        def _():
            spool_ref[...] = acc2[...]
            stats_ref[...] = acc[...]


def _k_fused_pool(uout, zout, res, bias1x, batch3d, lnw1x, lnb1x):
    m = NB
    return pl.pallas_call(
        _fused_pool_body,
        grid=(2 * NB,),
        in_specs=[
            pl.BlockSpec((1, R, D2), lambda b: (0, b % m, 0)),
            pl.BlockSpec((1, R, D2), lambda b: (1, b % m, 0)),
            pl.BlockSpec((NC, R, 16), lambda b: (0, b % m, 0)),
            pl.BlockSpec((R, D), lambda b: (b % m, 0)),
            pl.BlockSpec((1, D), lambda b: (0, 0)),
            pl.BlockSpec((1, 1, R), lambda b: (b % m, 0, 0)),
            pl.BlockSpec((1, D), lambda b: (0, 0)),
            pl.BlockSpec((1, D), lambda b: (0, 0)),
        ],
        out_specs=[
            pl.BlockSpec((R, D), lambda b: (b % m, 0)),
            pl.BlockSpec((R, D), lambda b: (b % m, 0)),
            pl.BlockSpec((G, D), lambda b: (0, 0)),
            pl.BlockSpec((G, 8), lambda b: (0, 0)),
        ],
        out_shape=[
            jax.ShapeDtypeStruct((N, D), jnp.float32),
            jax.ShapeDtypeStruct((NP32, D), jnp.float32),
            jax.ShapeDtypeStruct((G, D), jnp.float32),
            jax.ShapeDtypeStruct((G, 8), jnp.float32),
        ],
        scratch_shapes=[pltpu.VMEM((G, 8), jnp.float32),
                        pltpu.VMEM((G, D), jnp.float32)],
    )(uout, uout, zout, res, bias1x, batch3d, lnw1x, lnb1x)


def _s3_body(h_hbm, b_hbm, accz_hbm, mx_hbm, rows_v, bv_v, acc_v, sem):
    c = lax.axis_index("c")
    s = lax.axis_index("s")
    w = s * NC + c
    pltpu.sync_copy(accz_hbm, acc_v)
    base_w = w * (NP32 // NW)

    def chunk(g, carry):
        base = base_w + g * K
        pltpu.sync_copy(h_hbm.at[pl.ds(base, K)], rows_v)
        pltpu.sync_copy(b_hbm.at[pl.ds(base, K)], bv_v.at[pl.ds(0, K)])

        def node(e, cc):
            gsc = bv_v[pl.ds(e, L)][0]
            for k in range(D // L):
                cur = acc_v[gsc, pl.ds(k * L, L)]
                row = rows_v[e, pl.ds(k * L, L)]
                acc_v[gsc, pl.ds(k * L, L)] = jnp.maximum(cur, row)
            return cc

        lax.fori_loop(0, K, node, 0)
        return carry

    lax.fori_loop(0, NP32 // NW // K, chunk, 0)
    pltpu.sync_copy(acc_v, mx_hbm.at[w])


def _s3(h2p, batchp, accz):
    f = pl.kernel(
        _s3_body,
        out_type=[jax.ShapeDtypeStruct((NW, GP, D), jnp.float32)],
        mesh=plsc.VectorSubcoreMesh(core_axis_name="c", subcore_axis_name="s"),
        compiler_params=pltpu.CompilerParams(use_tc_tiling_on_sc=False),
        scratch_types=[
            pltpu.VMEM((K, D), jnp.float32),
            pltpu.VMEM((K + L,), jnp.int32),
            pltpu.VMEM((GP, D), jnp.float32),
            pltpu.SemaphoreType.DMA,
        ],
    )
    return f(h2p, batchp, accz)


def _erf(x):
    a1, a2, a3 = 0.254829592, -0.284496736, 1.421413741
    a4, a5, pp = -1.453152027, 1.061405429, 0.3275911
    ax = jnp.abs(x)
    t = 1.0 / (1.0 + pp * ax)
    y = 1.0 - (((((a5 * t + a4) * t) + a3) * t + a2) * t + a1) * t * jnp.exp(-ax * ax)
    return jnp.sign(x) * y


def _head_body(mx_ref, spool_ref, stats_ref, p1w_ref, p1b_ref,
               p2w_ref, p2b_ref, o_ref):
    mx = mx_ref[0]
    for i in range(1, NW):
        mx = jnp.maximum(mx, mx_ref[i])
    mx = mx[0:G, :]
    mx = jnp.where(mx > NEG * 0.5, mx, 0.0)
    cnt = jnp.maximum(stats_ref[...][:, 0:1], 1.0)
    mp = spool_ref[...] / cnt
    z1 = (jnp.dot(mp, p1w_ref[0:D, :], preferred_element_type=jnp.float32)
          + jnp.dot(mx, p1w_ref[D:2 * D, :], preferred_element_type=jnp.float32)
          + p1b_ref[...])
    gz = z1 * 0.5 * (1.0 + _erf(z1 * 0.7071067811865476))
    o = jnp.dot(gz, p2w_ref[...], preferred_element_type=jnp.float32) + p2b_ref[...]
    o = o / jnp.maximum(jnp.sqrt(jnp.sum(o * o, axis=-1, keepdims=True)), 1e-12)
    o_ref[...] = o


def _k_head(mxparts, spool, stats, p1w, p1b1x, p2w, p2b1x):
    return pl.pallas_call(
        _head_body,
        out_shape=jax.ShapeDtypeStruct((G, OUT), jnp.float32),
    )(mxparts, spool, stats, p1w, p1b1x, p2w, p2b1x)


def kernel(node_emb, W0, as0, ad0, b0, lnw0, lnb0, W1, as1, ad1, b1,
           lnw1, lnb1, p1w, p1b, p2w, p2b, x, edge_index, batch):
    i32 = jnp.int32
    x3d = x.astype(i32).reshape(NB, 1, R)
    batch3d = batch.astype(i32).reshape(NB, 1, R)
    loops = jnp.arange(N, dtype=i32)
    srcp = jnp.concatenate([edge_index[0].astype(i32), loops,
                            jnp.zeros((EP - E - N,), i32)])
    dstp = jnp.concatenate([edge_index[1].astype(i32), loops,
                            jnp.full((EP - E - N,), N, i32)])

    zz16 = jnp.zeros((NZP, 16), jnp.float32)
    zu = jnp.zeros((NZP, D2), jnp.float32)
    accz = jnp.full((GP, D), NEG, jnp.float32)

    h0, xw0, alr0 = _k1(x3d, node_emb, W0, as0.reshape(1, D), ad0.reshape(1, D))
    p0, zout0 = _s1(srcp, dstp, alr0, zz16)
    uout0 = _s2(srcp, dstp, p0, xw0.reshape(2 * N, D2), zu)[0]
    _, h1, xw1, alr1 = _k_fused_map(uout0, zout0, h0, b0.reshape(1, D),
                                    batch3d, lnw0.reshape(1, D),
                                    lnb0.reshape(1, D), W1, as1.reshape(1, D),
                                    ad1.reshape(1, D))

    p1, zout1 = _s1(srcp, dstp, alr1, zz16)
    uout1 = _s2(srcp, dstp, p1, xw1.reshape(2 * N, D2), zu)[0]
    _, h2, spool, stats1 = _k_fused_pool(uout1, zout1, h1, b1.reshape(1, D),
                                         batch3d, lnw1.reshape(1, D),
                                         lnb1.reshape(1, D))

    batchp = jnp.pad(batch.astype(i32), (0, NP32 - N), constant_values=G)
    mxparts = _s3(h2, batchp, accz)[0]
    return _k_head(mxparts, spool, stats1, p1w, p1b.reshape(1, OUT),
                   p2w, p2b.reshape(1, OUT))


# speedup vs baseline: 1.0757x; 1.0757x over previous
import jax
import jax.numpy as jnp
from jax import lax
from jax.experimental import pallas as pl
from jax.experimental.pallas import tpu as pltpu
from jax.experimental.pallas import tpu_sc as plsc

N = 50000
E = 800000
H = 4
C = 16
D = 64
G = 128
V = 256
OUT = 128

NC = 2
NS = 16
L = 16
NW = NC * NS

EP = 851968
K = 128
S1_CHUNKS = EP // NW // K
S2_CHUNKS = EP // NS // K

HALF = N // 2
HALFP = HALF + 88
NZP = N + 48
NP32 = 53248
GP = G + 8

D2 = D // 2
R = 1000
NB = N // R
NEG = -3.0e38


def _f32(v):
    return v.astype(jnp.float32)


def _head_sum_matrix():
    f = lax.broadcasted_iota(jnp.int32, (D, H), 0) // C
    hh = lax.broadcasted_iota(jnp.int32, (D, H), 1)
    return _f32(f == hh)


def _expand_matrix():
    hh = lax.broadcasted_iota(jnp.int32, (H, D), 0)
    f = lax.broadcasted_iota(jnp.int32, (H, D), 1) // C
    return _f32(f == hh)


def _col_place(n, m, at):
    i = lax.broadcasted_iota(jnp.int32, (n, m), 0)
    j = lax.broadcasted_iota(jnp.int32, (n, m), 1)
    return _f32(j == i + at)


def _xw_al_ar(h, w_ref, asf_ref, adf_ref):
    xw = jnp.dot(h, w_ref[...], preferred_element_type=jnp.float32)
    m = _head_sum_matrix()
    al = jnp.dot(xw * asf_ref[...], m, preferred_element_type=jnp.float32)
    ar = jnp.dot(xw * adf_ref[...], m, preferred_element_type=jnp.float32)
    alr = (jnp.dot(jnp.exp(al), _col_place(H, 16, 0),
                   preferred_element_type=jnp.float32)
           + jnp.dot(jnp.exp(0.2 * al), _col_place(H, 16, 4),
                     preferred_element_type=jnp.float32)
           + jnp.dot(jnp.exp(ar), _col_place(H, 16, 8),
                     preferred_element_type=jnp.float32)
           + jnp.dot(jnp.exp(0.2 * ar), _col_place(H, 16, 12),
                     preferred_element_type=jnp.float32))
    return xw, alr


def _k1_body(x_ref, emb_ref, w_ref, asf_ref, adf_ref, h_ref, xw_ref, alr_ref):
    ids = x_ref[0, 0, :]
    oh = _f32(ids[:, None] == lax.broadcasted_iota(jnp.int32, (R, V), 1))
    h = jnp.dot(oh, emb_ref[...], preferred_element_type=jnp.float32)
    xw, alr = _xw_al_ar(h, w_ref, asf_ref, adf_ref)
    h_ref[...] = h
    xw_ref[0] = xw[:, 0:D2]
    xw_ref[1] = xw[:, D2:D]
    alr_ref[...] = alr


def _k1(x3d, emb, w0, asf, adf):
    return pl.pallas_call(
        _k1_body,
        grid=(NB,),
        in_specs=[
            pl.BlockSpec((1, 1, R), lambda b: (b, 0, 0)),
            pl.BlockSpec((V, D), lambda b: (0, 0)),
            pl.BlockSpec((D, D), lambda b: (0, 0)),
            pl.BlockSpec((1, D), lambda b: (0, 0)),
            pl.BlockSpec((1, D), lambda b: (0, 0)),
        ],
        out_specs=[
            pl.BlockSpec((R, D), lambda b: (b, 0)),
            pl.BlockSpec((2, R, D2), lambda b: (0, b, 0)),
            pl.BlockSpec((R, 16), lambda b: (b, 0)),
        ],
        out_shape=[
            jax.ShapeDtypeStruct((N, D), jnp.float32),
            jax.ShapeDtypeStruct((2, N, D2), jnp.float32),
            jax.ShapeDtypeStruct((N, 16), jnp.float32),
        ],
    )(x3d, emb, w0, asf, adf)


def _vgather(v, idx16):
    return lax.gather(
        v, idx16[:, None],
        lax.GatherDimensionNumbers(offset_dims=(), collapsed_slice_dims=(0,),
                                   start_index_map=(0,)),
        (1,), mode=lax.GatherScatterMode.PROMISE_IN_BOUNDS)


S1_PAIRS = S1_CHUNKS // 2
S2_PAIRS = S2_CHUNKS // 2


def _s1_body(src_hbm, dst_hbm, alr_hbm, zz_hbm, p_hbm, zout_hbm,
             zsp,
             src_a, dst_a, dstc_a, ga_a, gb_a, p_a,
             src_b, dst_b, dstc_b, ga_b, gb_b, p_b,
             lsem_a, lsem_b, gsem_a, gsem_b):
    c = lax.axis_index("c")
    s = lax.axis_index("s")
    i16 = lax.iota(jnp.int32, L)
    shf8 = jnp.bitwise_and(i16 + 8, 15)
    shf4 = jnp.bitwise_and(i16 + 4, 15)
    zrows = NZP // NS
    pltpu.sync_copy(zz_hbm.at[pl.ds(s * zrows, zrows)],
                    zsp.at[pl.ds(s * zrows, zrows)])
    plsc.subcore_barrier()
    base_w = c * (EP // NC) + s * (EP // NW)

    A = (src_a, dst_a, dstc_a, ga_a, gb_a, p_a, lsem_a, gsem_a)
    B = (src_b, dst_b, dstc_b, ga_b, gb_b, p_b, lsem_b, gsem_b)

    def lin_issue(g, bufs):
        sv, dv = bufs[0], bufs[1]
        base = base_w + g * K
        pltpu.async_copy(src_hbm.at[pl.ds(base, K)], sv, bufs[6])
        pltpu.async_copy(dst_hbm.at[pl.ds(base, K)], dv, bufs[6])

    def lin_wait(g, bufs):
        sv, dv = bufs[0], bufs[1]
        base = base_w + g * K
        pltpu.make_async_copy(src_hbm.at[pl.ds(base, K)], sv, bufs[6]).wait()
        pltpu.make_async_copy(dst_hbm.at[pl.ds(base, K)], dv, bufs[6]).wait()

    def gat_issue(bufs):
        sv, dv, dcv, ga, gb = bufs[0], bufs[1], bufs[2], bufs[3], bufs[4]
        for i in range(K // L):
            dcv[pl.ds(i * L, L)] = jnp.minimum(dv[pl.ds(i * L, L)], N - 1)
        pltpu.async_copy(alr_hbm.at[sv], ga, bufs[7])
        pltpu.async_copy(alr_hbm.at[dcv], gb, bufs[7])

    def gat_wait(bufs):
        sv, dcv, ga, gb = bufs[0], bufs[2], bufs[3], bufs[4]
        pltpu.make_async_copy(alr_hbm.at[sv], ga, bufs[7]).wait()
        pltpu.make_async_copy(alr_hbm.at[dcv], gb, bufs[7]).wait()

    def compute(g, bufs):
        dv, ga, gb, pv = bufs[1], bufs[3], bufs[4], bufs[5]
        base = base_w + g * K

        def edge(e, cc):
            prod = ga[e, :] * _vgather(gb[e, :], shf8)
            pv[e, :] = jnp.maximum(prod, _vgather(prod, shf4))
            return cc

        lax.fori_loop(0, K, edge, 0, unroll=4)
        pltpu.sync_copy(pv, p_hbm.at[pl.ds(base, K)])
        pltpu.sync_copy(pv, zsp.at[dv], add=True)

    lin_issue(0, A)
    lin_wait(0, A)
    gat_issue(A)
    lin_issue(1, B)

    def pair(i, carry):
        g0 = 2 * i
        lin_wait(g0 + 1, B)
        gat_issue(B)
        gat_wait(A)
        compute(g0, A)
        lin_issue(g0 + 2, A)
        gat_wait(B)
        compute(g0 + 1, B)
        lin_wait(g0 + 2, A)
        gat_issue(A)
        lin_issue(g0 + 3, B)
        return carry

    lax.fori_loop(0, S1_PAIRS - 1, pair, 0)
    g0 = S1_CHUNKS - 2
    lin_wait(g0 + 1, B)
    gat_issue(B)
    gat_wait(A)
    compute(g0, A)
    gat_wait(B)
    compute(g0 + 1, B)

    plsc.subcore_barrier()
    pltpu.sync_copy(zsp.at[pl.ds(s * zrows, zrows)],
                    zout_hbm.at[c, pl.ds(s * zrows, zrows)])


def _s1(srcp, dstp, alr, zz16):
    f = pl.kernel(
        _s1_body,
        out_type=[
            jax.ShapeDtypeStruct((EP, 16), jnp.float32),
            jax.ShapeDtypeStruct((NC, NZP, 16), jnp.float32),
        ],
        mesh=plsc.VectorSubcoreMesh(core_axis_name="c", subcore_axis_name="s"),
        compiler_params=pltpu.CompilerParams(use_tc_tiling_on_sc=False),
        scratch_types=(
            [pltpu.VMEM_SHARED((NZP, 16), jnp.float32)]
            + 2 * [pltpu.VMEM((K,), jnp.int32),
                   pltpu.VMEM((K,), jnp.int32),
                   pltpu.VMEM((K,), jnp.int32),
                   pltpu.VMEM((K, 16), jnp.float32),
                   pltpu.VMEM((K, 16), jnp.float32),
                   pltpu.VMEM((K, 16), jnp.float32)]
            + 4 * [pltpu.SemaphoreType.DMA]
        ),
    )
    return f(srcp, dstp, alr, zz16)


def _s2_body(src_hbm, dst_hbm, p_hbm, xw_hbm, zu_hbm, uout_hbm,
             usp,
             src_a, dst_a, idxg_a, p_a, xw_a, siv_a,
             src_b, dst_b, idxg_b, p_b, xw_b, siv_b,
             lsem_a, lsem_b, gsem_a, gsem_b, ssem_a, ssem_b):
    c = lax.axis_index("c")
    s = lax.axis_index("s")
    urows = NZP // NS
    pltpu.sync_copy(zu_hbm.at[pl.ds(s * urows, urows)],
                    usp.at[pl.ds(s * urows, urows)])
    plsc.subcore_barrier()
    base_w = s * (EP // NS)
    rowoff = c * N

    A = (src_a, dst_a, idxg_a, p_a, xw_a, lsem_a, gsem_a, siv_a, ssem_a)
    B = (src_b, dst_b, idxg_b, p_b, xw_b, lsem_b, gsem_b, siv_b, ssem_b)

    def lin_issue(g, bufs):
        sv, dv, pv = bufs[0], bufs[1], bufs[3]
        base = base_w + g * K
        pltpu.async_copy(src_hbm.at[pl.ds(base, K)], sv, bufs[5])
        pltpu.async_copy(dst_hbm.at[pl.ds(base, K)], dv, bufs[5])
        pltpu.async_copy(p_hbm.at[pl.ds(base, K)], pv, bufs[5])

    def lin_wait(g, bufs):
        sv, dv, pv = bufs[0], bufs[1], bufs[3]
        base = base_w + g * K
        pltpu.make_async_copy(src_hbm.at[pl.ds(base, K)], sv, bufs[5]).wait()
        pltpu.make_async_copy(dst_hbm.at[pl.ds(base, K)], dv, bufs[5]).wait()
        pltpu.make_async_copy(p_hbm.at[pl.ds(base, K)], pv, bufs[5]).wait()

    def gat_issue(bufs):
        sv, gv = bufs[0], bufs[2]
        for i in range(K // L):
            gv[pl.ds(i * L, L)] = sv[pl.ds(i * L, L)] + rowoff
        pltpu.async_copy(xw_hbm.at[gv], bufs[4], bufs[6])

    def gat_wait(bufs):
        pltpu.make_async_copy(xw_hbm.at[bufs[2]], bufs[4], bufs[6]).wait()

    ph0 = jnp.full((L,), 2 * c, jnp.int32)
    ph1 = ph0 + 1

    def compute(bufs):
        dv, pv, xv, siv = bufs[1], bufs[3], bufs[4], bufs[7]
        for i in range(K // L):
            siv[pl.ds(i * L, L)] = dv[pl.ds(i * L, L)]

        def edge(e, cc):
            prow = pv[e, :]
            pk0 = _vgather(prow, ph0)
            pk1 = _vgather(prow, ph1)
            xv[e, pl.ds(0, L)] = xv[e, pl.ds(0, L)] * pk0
            xv[e, pl.ds(L, L)] = xv[e, pl.ds(L, L)] * pk1
            return cc

        lax.fori_loop(0, K, edge, 0, unroll=4)
        pltpu.async_copy(xv, usp.at[siv], bufs[8], add=True)

    def swait(bufs):
        pltpu.make_async_copy(bufs[4], usp.at[bufs[7]], bufs[8]).wait()

    lin_issue(0, A)
    lin_wait(0, A)
    gat_issue(A)
    lin_issue(1, B)

    lin_wait(1, B)
    gat_issue(B)
    gat_wait(A)
    compute(A)
    lin_issue(2, A)
    gat_wait(B)
    compute(B)
    lin_wait(2, A)
    swait(A)
    gat_issue(A)
    lin_issue(3, B)

    def pair(i, carry):
        g0 = 2 * i
        lin_wait(g0 + 1, B)
        swait(B)
        gat_issue(B)
        gat_wait(A)
        compute(A)
        lin_issue(g0 + 2, A)
        gat_wait(B)
        compute(B)
        lin_wait(g0 + 2, A)
        swait(A)
        gat_issue(A)
        lin_issue(g0 + 3, B)
        return carry

    lax.fori_loop(1, S2_PAIRS - 1, pair, 0)
    lin_wait(S2_CHUNKS - 1, B)
    swait(B)
    gat_issue(B)
    gat_wait(A)
    compute(A)
    gat_wait(B)
    compute(B)
    swait(A)
    swait(B)

    plsc.subcore_barrier()
    pltpu.sync_copy(usp.at[pl.ds(s * urows, urows)],
                    uout_hbm.at[c, pl.ds(s * urows, urows)])


def _s2(srcp, dstp, p, xw2, zu):
    f = pl.kernel(
        _s2_body,
        out_type=[
            jax.ShapeDtypeStruct((NC, NZP, D2), jnp.float32),
        ],
        mesh=plsc.VectorSubcoreMesh(core_axis_name="c", subcore_axis_name="s"),
        compiler_params=pltpu.CompilerParams(use_tc_tiling_on_sc=False),
        scratch_types=(
            [pltpu.VMEM_SHARED((NZP, D2), jnp.float32)]
            + 2 * [pltpu.VMEM((K,), jnp.int32),
                   pltpu.VMEM((K,), jnp.int32),
                   pltpu.VMEM((K,), jnp.int32),
                   pltpu.VMEM((K, 16), jnp.float32),
                   pltpu.VMEM((K, D2), jnp.float32),
                   pltpu.VMEM((K,), jnp.int32)]
            + 6 * [pltpu.SemaphoreType.DMA]
        ),
    )
    return f(srcp, dstp, p, xw2, zu)


def _stats_body(u0_ref, u1_ref, z_ref, res_ref, bias_ref, batch_ref,
                t_ref, stats_ref, acc):
    b = pl.program_id(0)
    zs = z_ref[0, :, 0:H] + z_ref[1, :, 0:H]
    rz = 1.0 / (zs + 1e-16)
    rzx = jnp.dot(rz, _expand_matrix(), preferred_element_type=jnp.float32)
    t0 = u0_ref[0] * rzx[:, 0:D2] + bias_ref[..., 0:D2] + res_ref[:, 0:D2]
    t1 = u1_ref[0] * rzx[:, D2:D] + bias_ref[..., D2:D] + res_ref[:, D2:D]
    t_ref[:, 0:D2] = t0
    t_ref[:, D2:D] = t1
    t = jnp.concatenate([t0, t1], axis=1)

    ids = batch_ref[0, 0, :]
    oh = _f32(ids[:, None] == lax.broadcasted_iota(jnp.int32, (R, G), 1))
    ones = jnp.ones((R, 1), jnp.float32)
    a = jnp.sum(t, axis=1, keepdims=True)
    q = jnp.sum(t * t, axis=1, keepdims=True)
    cols = (jnp.dot(ones, _col_place(1, 8, 0), preferred_element_type=jnp.float32)
            + jnp.dot(a, _col_place(1, 8, 1), preferred_element_type=jnp.float32)
            + jnp.dot(q, _col_place(1, 8, 2), preferred_element_type=jnp.float32))
    blk = lax.dot_general(oh, cols, (((0,), (0,)), ((), ())),
                          preferred_element_type=jnp.float32)

    @pl.when(b == 0)
    def _():
        acc[...] = jnp.zeros_like(acc)

    acc[...] += blk

    @pl.when(b == NB - 1)
    def _():
        stats_ref[...] = acc[...]


def _k_stats(uout, zout, res, bias1x, batch3d):
    return pl.pallas_call(
        _stats_body,
        grid=(NB,),
        in_specs=[
            pl.BlockSpec((1, R, D2), lambda b: (0, b, 0)),
            pl.BlockSpec((1, R, D2), lambda b: (1, b, 0)),
            pl.BlockSpec((NC, R, 16), lambda b: (0, b, 0)),
            pl.BlockSpec((R, D), lambda b: (b, 0)),
            pl.BlockSpec((1, D), lambda b: (0, 0)),
            pl.BlockSpec((1, 1, R), lambda b: (b, 0, 0)),
        ],
        out_specs=[
            pl.BlockSpec((R, D), lambda b: (b, 0)),
            pl.BlockSpec((G, 8), lambda b: (0, 0)),
        ],
        out_shape=[
            jax.ShapeDtypeStruct((N, D), jnp.float32),
            jax.ShapeDtypeStruct((G, 8), jnp.float32),
        ],
        scratch_shapes=[pltpu.VMEM((G, 8), jnp.float32)],
    )(uout, uout, zout, res, bias1x, batch3d)


def _ln_rows(t, ids, stats_ref, lnw_ref, lnb_ref):
    st = stats_ref[...]
    cnt = st[:, 0:1]
    norm = jnp.maximum(cnt, 1.0) * D
    mean = st[:, 1:2] / norm
    var = st[:, 2:3] / norm - mean * mean
    inv = 1.0 / jnp.sqrt(var + 1e-5)
    gv = (jnp.dot(mean, _col_place(1, 8, 0), preferred_element_type=jnp.float32)
          + jnp.dot(inv, _col_place(1, 8, 1), preferred_element_type=jnp.float32))
    oh = _f32(ids[:, None] == lax.broadcasted_iota(jnp.int32, (R, G), 1))
    mv = jnp.dot(oh, gv, preferred_element_type=jnp.float32)
    return ((t - mv[:, 0:1]) * mv[:, 1:2]) * lnw_ref[...] + lnb_ref[...], oh


def _ln_map_body(t_ref, batch_ref, stats_ref, lnw_ref, lnb_ref,
                 w_ref, asf_ref, adf_ref, h_ref, xw_ref, alr_ref):
    hn, _ = _ln_rows(t_ref[...], batch_ref[0, 0, :], stats_ref, lnw_ref, lnb_ref)
    xw, alr = _xw_al_ar(hn, w_ref, asf_ref, adf_ref)
    h_ref[...] = hn
    xw_ref[0] = xw[:, 0:D2]
    xw_ref[1] = xw[:, D2:D]
    alr_ref[...] = alr


def _k_ln_map(t, batch3d, stats, lnw1x, lnb1x, w1, asf, adf):
    return pl.pallas_call(
        _ln_map_body,
        grid=(NB,),
        in_specs=[
            pl.BlockSpec((R, D), lambda b: (b, 0)),
            pl.BlockSpec((1, 1, R), lambda b: (b, 0, 0)),
            pl.BlockSpec((G, 8), lambda b: (0, 0)),
            pl.BlockSpec((1, D), lambda b: (0, 0)),
            pl.BlockSpec((1, D), lambda b: (0, 0)),
            pl.BlockSpec((D, D), lambda b: (0, 0)),
            pl.BlockSpec((1, D), lambda b: (0, 0)),
            pl.BlockSpec((1, D), lambda b: (0, 0)),
        ],
        out_specs=[
            pl.BlockSpec((R, D), lambda b: (b, 0)),
            pl.BlockSpec((2, R, D2), lambda b: (0, b, 0)),
            pl.BlockSpec((R, 16), lambda b: (b, 0)),
        ],
        out_shape=[
            jax.ShapeDtypeStruct((N, D), jnp.float32),
            jax.ShapeDtypeStruct((2, N, D2), jnp.float32),
            jax.ShapeDtypeStruct((N, 16), jnp.float32),
        ],
    )(t, batch3d, stats, lnw1x, lnb1x, w1, asf, adf)


def _ln_pool_body(t_ref, batch_ref, stats_ref, lnw_ref, lnb_ref,
                  h_ref, spool_ref, acc):
    b = pl.program_id(0)
    hn, oh = _ln_rows(t_ref[...], batch_ref[0, 0, :], stats_ref,
                      lnw_ref, lnb_ref)
    h_ref[...] = hn
    blk = lax.dot_general(oh, hn, (((0,), (0,)), ((), ())),
                          preferred_element_type=jnp.float32)

    @pl.when(b == 0)
    def _():
        acc[...] = jnp.zeros_like(acc)

    acc[...] += blk

    @pl.when(b == NB - 1)
    def _():
        spool_ref[...] = acc[...]


def _k_ln_pool(t, batch3d, stats, lnw1x, lnb1x):
    return pl.pallas_call(
        _ln_pool_body,
        grid=(NB,),
        in_specs=[
            pl.BlockSpec((R, D), lambda b: (b, 0)),
            pl.BlockSpec((1, 1, R), lambda b: (b, 0, 0)),
            pl.BlockSpec((G, 8), lambda b: (0, 0)),
            pl.BlockSpec((1, D), lambda b: (0, 0)),
            pl.BlockSpec((1, D), lambda b: (0, 0)),
        ],
        out_specs=[
            pl.BlockSpec((R, D), lambda b: (b, 0)),
            pl.BlockSpec((G, D), lambda b: (0, 0)),
        ],
        out_shape=[
            jax.ShapeDtypeStruct((NP32, D), jnp.float32),
            jax.ShapeDtypeStruct((G, D), jnp.float32),
        ],
        scratch_shapes=[pltpu.VMEM((G, D), jnp.float32)],
    )(t, batch3d, stats, lnw1x, lnb1x)


def _s3_body(h_hbm, b_hbm, accz_hbm, mx_hbm, rows_v, bv_v, acc_v, sem):
    c = lax.axis_index("c")
    s = lax.axis_index("s")
    w = s * NC + c
    pltpu.sync_copy(accz_hbm, acc_v)
    base_w = w * (NP32 // NW)

    def chunk(g, carry):
        base = base_w + g * K
        pltpu.sync_copy(h_hbm.at[pl.ds(base, K)], rows_v)
        pltpu.sync_copy(b_hbm.at[pl.ds(base, K)], bv_v.at[pl.ds(0, K)])

        def node(e, cc):
            gsc = bv_v[pl.ds(e, L)][0]
            for k in range(D // L):
                cur = acc_v[gsc, pl.ds(k * L, L)]
                row = rows_v[e, pl.ds(k * L, L)]
                acc_v[gsc, pl.ds(k * L, L)] = jnp.maximum(cur, row)
            return cc

        lax.fori_loop(0, K, node, 0)
        return carry

    lax.fori_loop(0, NP32 // NW // K, chunk, 0)
    pltpu.sync_copy(acc_v, mx_hbm.at[w])


def _s3(h2p, batchp, accz):
    f = pl.kernel(
        _s3_body,
        out_type=[jax.ShapeDtypeStruct((NW, GP, D), jnp.float32)],
        mesh=plsc.VectorSubcoreMesh(core_axis_name="c", subcore_axis_name="s"),
        compiler_params=pltpu.CompilerParams(use_tc_tiling_on_sc=False),
        scratch_types=[
            pltpu.VMEM((K, D), jnp.float32),
            pltpu.VMEM((K + L,), jnp.int32),
            pltpu.VMEM((GP, D), jnp.float32),
            pltpu.SemaphoreType.DMA,
        ],
    )
    return f(h2p, batchp, accz)


def _erf(x):
    a1, a2, a3 = 0.254829592, -0.284496736, 1.421413741
    a4, a5, pp = -1.453152027, 1.061405429, 0.3275911
    ax = jnp.abs(x)
    t = 1.0 / (1.0 + pp * ax)
    y = 1.0 - (((((a5 * t + a4) * t) + a3) * t + a2) * t + a1) * t * jnp.exp(-ax * ax)
    return jnp.sign(x) * y


def _head_body(mx_ref, spool_ref, stats_ref, p1w_ref, p1b_ref,
               p2w_ref, p2b_ref, o_ref):
    mx = mx_ref[0]
    for i in range(1, NW):
        mx = jnp.maximum(mx, mx_ref[i])
    mx = mx[0:G, :]
    mx = jnp.where(mx > NEG * 0.5, mx, 0.0)
    cnt = jnp.maximum(stats_ref[...][:, 0:1], 1.0)
    mp = spool_ref[...] / cnt
    z1 = (jnp.dot(mp, p1w_ref[0:D, :], preferred_element_type=jnp.float32)
          + jnp.dot(mx, p1w_ref[D:2 * D, :], preferred_element_type=jnp.float32)
          + p1b_ref[...])
    gz = z1 * 0.5 * (1.0 + _erf(z1 * 0.7071067811865476))
    o = jnp.dot(gz, p2w_ref[...], preferred_element_type=jnp.float32) + p2b_ref[...]
    o = o / jnp.maximum(jnp.sqrt(jnp.sum(o * o, axis=-1, keepdims=True)), 1e-12)
    o_ref[...] = o


def _k_head(mxparts, spool, stats, p1w, p1b1x, p2w, p2b1x):
    return pl.pallas_call(
        _head_body,
        out_shape=jax.ShapeDtypeStruct((G, OUT), jnp.float32),
    )(mxparts, spool, stats, p1w, p1b1x, p2w, p2b1x)


def kernel(node_emb, W0, as0, ad0, b0, lnw0, lnb0, W1, as1, ad1, b1,
           lnw1, lnb1, p1w, p1b, p2w, p2b, x, edge_index, batch):
    i32 = jnp.int32
    x3d = x.astype(i32).reshape(NB, 1, R)
    batch3d = batch.astype(i32).reshape(NB, 1, R)
    loops = jnp.arange(N, dtype=i32)
    srcp = jnp.concatenate([edge_index[0].astype(i32), loops,
                            jnp.zeros((EP - E - N,), i32)])
    dstp = jnp.concatenate([edge_index[1].astype(i32), loops,
                            jnp.full((EP - E - N,), N, i32)])

    zz16 = jnp.zeros((NZP, 16), jnp.float32)
    zu = jnp.zeros((NZP, D2), jnp.float32)
    accz = jnp.full((GP, D), NEG, jnp.float32)

    h0, xw0, alr0 = _k1(x3d, node_emb, W0, as0.reshape(1, D), ad0.reshape(1, D))
    p0, zout0 = _s1(srcp, dstp, alr0, zz16)
    uout0 = _s2(srcp, dstp, p0, xw0.reshape(2 * N, D2), zu)[0]
    t0, stats0 = _k_stats(uout0, zout0, h0, b0.reshape(1, D), batch3d)
    h1, xw1, alr1 = _k_ln_map(t0, batch3d, stats0, lnw0.reshape(1, D),
                              lnb0.reshape(1, D), W1, as1.reshape(1, D),
                              ad1.reshape(1, D))

    p1, zout1 = _s1(srcp, dstp, alr1, zz16)
    uout1 = _s2(srcp, dstp, p1, xw1.reshape(2 * N, D2), zu)[0]
    t1, stats1 = _k_stats(uout1, zout1, h1, b1.reshape(1, D), batch3d)
    h2, spool = _k_ln_pool(t1, batch3d, stats1, lnw1.reshape(1, D),
                           lnb1.reshape(1, D))

    batchp = jnp.pad(batch.astype(i32), (0, NP32 - N), constant_values=G)
    mxparts = _s3(h2, batchp, accz)[0]
    return _k_head(mxparts, spool, stats1, p1w, p1b.reshape(1, OUT),
                   p2w, p2b.reshape(1, OUT))
